# Initial kernel scaffold; baseline (speedup 1.0000x reference)
import jax
import jax.numpy as jnp
from jax.experimental import pallas as pl


def kernel(z, Wg, bg, W1, b1, W2, b2, W3, b3):
    raise NotImplementedError("write your pallas kernel here")


# speedup vs baseline: 602.8684x; 602.8684x over previous
import functools

import numpy as np
import jax
import jax.numpy as jnp
from jax import lax
from jax.experimental import pallas as pl
from jax.experimental.pallas import tpu as pltpu
from jax.experimental.pallas import tpu_sc as plsc

_TABLE_NUM = 16
_TABLE_SIZE = 8192
_IMG = 256
_N = _IMG * _IMG
_BATCH = 4
_FEAT = 2
_HID = 64


def _resolutions():
    b = np.exp((np.log(256.0) - np.log(16.0)) / (_TABLE_NUM - 1))
    return np.floor(16.0 * (b ** np.arange(_TABLE_NUM))).astype(np.int64)


def _build_consts():
    res = _resolutions()
    pi2 = np.uint32(2654435761)
    c = (np.arange(_IMG, dtype=np.float32) + np.float32(0.5)) / np.float32(_IMG)
    h00 = np.empty((_TABLE_NUM, _N), np.int32)
    h01 = np.empty((_TABLE_NUM, _N), np.int32)
    h10 = np.empty((_TABLE_NUM, _N), np.int32)
    h11 = np.empty((_TABLE_NUM, _N), np.int32)
    fx = np.empty((_TABLE_NUM, _N), np.float32)
    fy = np.empty((_TABLE_NUM, _N), np.float32)
    for lvl in range(_TABLE_NUM):
        r = np.float32(float(res[lvl]))
        xy = c * r
        x0 = np.floor(xy)
        fr = (xy - x0).astype(np.float32)
        i0 = x0.astype(np.uint32)
        i1 = i0 + np.uint32(1)
        ix0 = i0[:, None]
        ix1 = i1[:, None]
        m0 = (i0 * pi2)[None, :]
        m1 = (i1 * pi2)[None, :]

        def hsh(a, m):
            return ((a ^ m) % np.uint32(_TABLE_SIZE)).astype(np.int32)

        h00[lvl] = hsh(ix0, m0).ravel()
        h01[lvl] = hsh(ix0, m1).ravel()
        h10[lvl] = hsh(ix1, m0).ravel()
        h11[lvl] = hsh(ix1, m1).ravel()
        fx[lvl] = np.broadcast_to(fr[:, None], (_IMG, _IMG)).ravel()
        fy[lvl] = np.broadcast_to(fr[None, :], (_IMG, _IMG)).ravel()
    return h00, h01, h10, h11, fx, fy


_H00, _H01, _H10, _H11, _FX, _FY = _build_consts()


_CB = 8192


def _tablegen_body(z_ref, wg_ref, bg_ref, out_ref):
    acc = jnp.dot(z_ref[...], wg_ref[...], preferred_element_type=jnp.float32)
    out_ref[...] = jnp.tanh(acc + bg_ref[...])


def _tablegen(z, Wg, bg):
    ncols = Wg.shape[1]
    return pl.pallas_call(
        _tablegen_body,
        grid=(ncols // _CB,),
        in_specs=[
            pl.BlockSpec((_BATCH, 64), lambda i: (0, 0)),
            pl.BlockSpec((64, _CB), lambda i: (0, i)),
            pl.BlockSpec((1, _CB), lambda i: (0, i)),
        ],
        out_specs=pl.BlockSpec((_BATCH, _CB), lambda i: (0, i)),
        out_shape=jax.ShapeDtypeStruct((_BATCH, ncols), jnp.float32),
    )(z, Wg, bg.reshape(1, -1))


_NC = 2
_NS = 16
_CHUNK = 4096


@functools.cache
def _make_sc_gather():
    return functools.partial(
        pl.kernel,
        mesh=plsc.VectorSubcoreMesh(core_axis_name="c", subcore_axis_name="s"),
        compiler_params=pltpu.CompilerParams(needs_layout_passes=False),
        out_type=jax.ShapeDtypeStruct((_BATCH, 2 * _TABLE_NUM, _N), jnp.float32),
        scratch_types=[
            pltpu.VMEM((2 * _TABLE_SIZE,), jnp.float32),
            pltpu.VMEM((_CHUNK,), jnp.int32),
            pltpu.VMEM((_CHUNK,), jnp.int32),
            pltpu.VMEM((_CHUNK,), jnp.int32),
            pltpu.VMEM((_CHUNK,), jnp.int32),
            pltpu.VMEM((_CHUNK,), jnp.float32),
            pltpu.VMEM((_CHUNK,), jnp.float32),
            pltpu.VMEM((2, _CHUNK), jnp.float32),
        ],
    )(_sc_gather_body)


def _sc_gather_body(tables, h00, h01, h10, h11, fx, fy, featT,
                    table_v, h00_v, h01_v, h10_v, h11_v, fx_v, fy_v, stage_v):
    wid = lax.axis_index("s") * _NC + lax.axis_index("c")
    for t in range(2):
        task = wid * 2 + t
        b = task // _TABLE_NUM
        lvl = task % _TABLE_NUM
        pltpu.sync_copy(tables.at[b, lvl], table_v)

        def chunk_body(ci, carry):
            off = ci * _CHUNK
            pltpu.sync_copy(h00.at[lvl, pl.ds(off, _CHUNK)], h00_v)
            pltpu.sync_copy(h01.at[lvl, pl.ds(off, _CHUNK)], h01_v)
            pltpu.sync_copy(h10.at[lvl, pl.ds(off, _CHUNK)], h10_v)
            pltpu.sync_copy(h11.at[lvl, pl.ds(off, _CHUNK)], h11_v)
            pltpu.sync_copy(fx.at[lvl, pl.ds(off, _CHUNK)], fx_v)
            pltpu.sync_copy(fy.at[lvl, pl.ds(off, _CHUNK)], fy_v)

            def grp(g, c2):
                s = g * 16
                i00 = h00_v[pl.ds(s, 16)] * 2
                i01 = h01_v[pl.ds(s, 16)] * 2
                i10 = h10_v[pl.ds(s, 16)] * 2
                i11 = h11_v[pl.ds(s, 16)] * 2
                f00a = plsc.load_gather(table_v, [i00])
                f01a = plsc.load_gather(table_v, [i01])
                f10a = plsc.load_gather(table_v, [i10])
                f11a = plsc.load_gather(table_v, [i11])
                f00b = plsc.load_gather(table_v, [i00 + 1])
                f01b = plsc.load_gather(table_v, [i01 + 1])
                f10b = plsc.load_gather(table_v, [i10 + 1])
                f11b = plsc.load_gather(table_v, [i11 + 1])
                fxv = fx_v[pl.ds(s, 16)]
                fyv = fy_v[pl.ds(s, 16)]
                c0a = f00a + fxv * (f10a - f00a)
                c1a = f01a + fxv * (f11a - f01a)
                c0b = f00b + fxv * (f10b - f00b)
                c1b = f01b + fxv * (f11b - f01b)
                stage_v[0, pl.ds(s, 16)] = c0a + fyv * (c1a - c0a)
                stage_v[1, pl.ds(s, 16)] = c0b + fyv * (c1b - c0b)
                return c2

            lax.fori_loop(0, _CHUNK // 16, grp, 0)
            pltpu.sync_copy(
                stage_v, featT.at[b, pl.ds(lvl * 2, 2), pl.ds(off, _CHUNK)])
            return carry

        lax.fori_loop(0, _N // _CHUNK, chunk_body, 0)


_TN = 2048


def _mlp_body(x_ref, w1_ref, b1_ref, w2_ref, b2_ref, w3_ref, b3_ref, out_ref):
    x = x_ref[0]
    h = jnp.dot(w1_ref[...], x, preferred_element_type=jnp.float32)
    h = jnp.maximum(h + b1_ref[...], 0.0)
    h = jnp.dot(w2_ref[...], h, preferred_element_type=jnp.float32)
    h = jnp.maximum(h + b2_ref[...], 0.0)
    o = jnp.dot(w3_ref[...], h, preferred_element_type=jnp.float32)
    out_ref[0] = jnp.tanh(o + b3_ref[...])


def _mlp(featT, W1T, b1, W2T, b2, W3T, b3):
    return pl.pallas_call(
        _mlp_body,
        grid=(_BATCH, _N // _TN),
        in_specs=[
            pl.BlockSpec((1, 2 * _TABLE_NUM, _TN), lambda b, i: (b, 0, i)),
            pl.BlockSpec((_HID, 2 * _TABLE_NUM), lambda b, i: (0, 0)),
            pl.BlockSpec((_HID, 1), lambda b, i: (0, 0)),
            pl.BlockSpec((_HID, _HID), lambda b, i: (0, 0)),
            pl.BlockSpec((_HID, 1), lambda b, i: (0, 0)),
            pl.BlockSpec((3, _HID), lambda b, i: (0, 0)),
            pl.BlockSpec((3, 1), lambda b, i: (0, 0)),
        ],
        out_specs=pl.BlockSpec((1, 3, _TN), lambda b, i: (b, 0, i)),
        out_shape=jax.ShapeDtypeStruct((_BATCH, 3, _N), jnp.float32),
    )(featT, W1T, b1.reshape(-1, 1), W2T, b2.reshape(-1, 1),
      W3T, b3.reshape(-1, 1))


def kernel(z, Wg, bg, W1, b1, W2, b2, W3, b3):
    tables_flat = _tablegen(z, Wg, bg)
    tables = tables_flat.reshape(_BATCH, _TABLE_NUM, 2 * _TABLE_SIZE)
    featT = _make_sc_gather()(tables, _H00, _H01, _H10, _H11, _FX, _FY)
    out = _mlp(featT, W1.T, b1, W2.T, b2, W3.T, b3)
    return out.reshape(_BATCH, 3, _IMG, _IMG)
